# Initial kernel scaffold; baseline (speedup 1.0000x reference)
import jax
import jax.numpy as jnp
from jax.experimental import pallas as pl


def kernel(params, x, edge_index, edge_attr, batch):
    raise NotImplementedError("write your pallas kernel here")


# speedup vs baseline: 32.1717x; 32.1717x over previous
import functools
import jax
import jax.numpy as jnp
from jax import lax
from jax.experimental import pallas as pl
from jax.experimental.pallas import tpu as pltpu

N = 50000
E = 800000
G = 256
L = 4
T = 4
F = 32
FO = 8
TF = T * F
BN = 1000
GRID = N // BN


def _prep_weights(params):
    w = {}
    for l in range(L):
        Wpre = params['Wpre%d' % l]
        w['Wd%d' % l] = jnp.transpose(Wpre[:, 0:32, :], (1, 0, 2)).reshape(32, TF)
        w['Ws%d' % l] = jnp.transpose(Wpre[:, 32:64, :], (1, 0, 2)).reshape(32, TF)
        Wel = jnp.transpose(Wpre[:, 64:96, :], (1, 0, 2)).reshape(32, TF)
        Te = params['edge_emb'] @ params['We%d' % l] + params['be%d' % l]
        w['Ttab%d' % l] = Te @ Wel + params['bpre%d' % l].reshape(TF)
        Wpost = params['Wpost%d' % l]
        w['Wx%d' % l] = jnp.transpose(Wpost[:, 0:F, :], (1, 0, 2)).reshape(F, T * FO)
        cols = []
        for v in range(3):
            rows = []
            for a in range(4):
                c = 1 + v * 4 + a
                blk = jax.scipy.linalg.block_diag(
                    *[Wpost[t, 32 * c:32 * c + 32, :] for t in range(T)])
                rows.append(blk)
            cols.append(jnp.concatenate(rows, axis=0))
        w['BD%d' % l] = jnp.concatenate(cols, axis=1)
        w['bpost%d' % l] = params['bpost%d' % l].reshape(1, T * FO)
        w['Wlin%d' % l] = params['Wlin%d' % l]
        w['blin%d' % l] = params['blin%d' % l].reshape(1, 32)
        w['gamma%d' % l] = params['gamma%d' % l].reshape(1, 32)
        w['beta%d' % l] = params['beta%d' % l].reshape(1, 32)
    emb = jnp.zeros((24, 32), jnp.float32).at[:23].set(params['node_emb'])
    w['emb'] = emb
    W1p = jnp.zeros((32, 56), jnp.float32).at[:, :50].set(params['W1'])
    b1p = jnp.zeros((1, 56), jnp.float32).at[0, :50].set(params['b1'])
    W2p = jnp.zeros((56, 32), jnp.float32).at[:50, :25].set(params['W2'])
    b2p = jnp.zeros((1, 32), jnp.float32).at[0, :25].set(params['b2'])
    W3p = jnp.zeros((32, 8), jnp.float32).at[:25, 0:1].set(params['W3'])
    b3p = jnp.zeros((1, 8), jnp.float32).at[0, 0].set(params['b3'][0])
    w.update(W1p=W1p, b1p=b1p, W2p=W2p, b2p=b2p, W3p=W3p, b3p=b3p)
    return w


def _k_embed_proj(x_ref, emb_ref, wd_ref, ws_ref, h_ref, a_ref, b_ref):
    xi = x_ref[...]
    oh = (xi == lax.broadcasted_iota(jnp.int32, (1, 24), 1)).astype(jnp.float32)
    h = jnp.dot(oh, emb_ref[...], preferred_element_type=jnp.float32, precision=lax.Precision.HIGHEST)
    h_ref[...] = h
    a_ref[...] = jnp.dot(h, wd_ref[...], preferred_element_type=jnp.float32, precision=lax.Precision.HIGHEST)
    b_ref[...] = jnp.dot(h, ws_ref[...], preferred_element_type=jnp.float32, precision=lax.Precision.HIGHEST)


def _k_bn_proj(out_ref_in, acc_ref, g_ref, be_ref, wd_ref, ws_ref,
               h_ref, a_ref, b_ref):
    s = acc_ref[0:1, :]
    sq = acc_ref[1:2, :]
    mu = s / N
    var = sq / N - mu * mu
    rs = lax.rsqrt(var + 1e-5)
    h = (out_ref_in[...] - mu) * rs * g_ref[...] + be_ref[...]
    h = jnp.maximum(h, 0.0)
    h_ref[...] = h
    a_ref[...] = jnp.dot(h, wd_ref[...], preferred_element_type=jnp.float32, precision=lax.Precision.HIGHEST)
    b_ref[...] = jnp.dot(h, ws_ref[...], preferred_element_type=jnp.float32, precision=lax.Precision.HIGHEST)


def _k_node(h_ref, a_ref, s1_ref, s2_ref, mn_ref, mx_ref, deg_ref, sc_ref,
            wx_ref, bd_ref, bpost_ref, wlin_ref, blin_ref,
            out_ref, acc_ref):
    i = pl.program_id(0)
    deg = deg_ref[...]
    degc = jnp.maximum(deg, 1.0)
    inv = 1.0 / degc
    mask = deg > 0.0
    A = a_ref[...]
    S1 = s1_ref[...]
    S2 = s2_ref[...]
    mean = jnp.where(mask, (deg * A + S1) * inv, 0.0)
    msq = jnp.where(mask, (deg * A * A + 2.0 * A * S1 + S2) * inv, 0.0)
    std = jnp.sqrt(jnp.maximum(msq - mean * mean, 0.0) + 1e-5)
    mn = jnp.where(mask, A + mn_ref[...], 0.0)
    mx = jnp.where(mask, A + mx_ref[...], 0.0)
    avg_log = sc_ref[0, 0]
    logd = jnp.log(degc + 1.0)
    amp = logd / avg_log
    att = avg_log / logd
    Gm = jnp.concatenate([mean, mn, mx, std], axis=1)
    P = jnp.dot(Gm, bd_ref[...], preferred_element_type=jnp.float32, precision=lax.Precision.HIGHEST)
    out = (jnp.dot(h_ref[...], wx_ref[...], preferred_element_type=jnp.float32, precision=lax.Precision.HIGHEST)
           + P[:, 0:32] + amp * P[:, 32:64] + att * P[:, 64:96]
           + bpost_ref[...])
    out = jnp.dot(out, wlin_ref[...], preferred_element_type=jnp.float32, precision=lax.Precision.HIGHEST) \
        + blin_ref[...]
    out_ref[...] = out

    @pl.when(i == 0)
    def _():
        acc_ref[...] = jnp.zeros_like(acc_ref)

    bs = jnp.sum(out, axis=0, keepdims=True)
    bq = jnp.sum(out * out, axis=0, keepdims=True)
    acc_ref[0:1, :] += bs
    acc_ref[1:2, :] += bq


def _k_final(out_ref_in, acc_ref, g_ref, be_ref, batch_ref,
             w1_ref, b1_ref, w2_ref, b2_ref, w3_ref, b3_ref,
             res_ref, pool_ref):
    i = pl.program_id(0)
    s = acc_ref[0:1, :]
    sq = acc_ref[1:2, :]
    mu = s / N
    var = sq / N - mu * mu
    rs = lax.rsqrt(var + 1e-5)
    h = (out_ref_in[...] - mu) * rs * g_ref[...] + be_ref[...]
    h = jnp.maximum(h, 0.0)
    bt = batch_ref[...].reshape(1, BN)
    ohT = (lax.broadcasted_iota(jnp.int32, (G, 1), 0) == bt).astype(jnp.float32)

    @pl.when(i == 0)
    def _():
        pool_ref[...] = jnp.zeros_like(pool_ref)

    pool_ref[...] += jnp.dot(ohT, h, preferred_element_type=jnp.float32, precision=lax.Precision.HIGHEST)

    @pl.when(i == GRID - 1)
    def _():
        pooled = pool_ref[...]
        z1 = jnp.maximum(
            jnp.dot(pooled, w1_ref[...], preferred_element_type=jnp.float32, precision=lax.Precision.HIGHEST)
            + b1_ref[...], 0.0)
        z2 = jnp.maximum(
            jnp.dot(z1, w2_ref[...], preferred_element_type=jnp.float32, precision=lax.Precision.HIGHEST)
            + b2_ref[...], 0.0)
        res_ref[...] = jnp.dot(z2, w3_ref[...],
                               preferred_element_type=jnp.float32, precision=lax.Precision.HIGHEST) + b3_ref[...]


def _bspec(shape, bcast=False):
    nd = len(shape)
    if bcast:
        return pl.BlockSpec(shape, lambda i: (0,) * nd)
    return pl.BlockSpec(shape, lambda i: (i,) + (0,) * (nd - 1))


def _embed_proj(x2, emb, wd, ws):
    return pl.pallas_call(
        _k_embed_proj,
        grid=(GRID,),
        in_specs=[_bspec((BN, 1)), _bspec((24, 32), True),
                  _bspec((32, TF), True), _bspec((32, TF), True)],
        out_specs=[_bspec((BN, 32)), _bspec((BN, TF)), _bspec((BN, TF))],
        out_shape=[jax.ShapeDtypeStruct((N, 32), jnp.float32),
                   jax.ShapeDtypeStruct((N, TF), jnp.float32),
                   jax.ShapeDtypeStruct((N, TF), jnp.float32)],
    )(x2, emb, wd, ws)


def _bn_proj(out_prev, acc, gamma, beta, wd, ws):
    return pl.pallas_call(
        _k_bn_proj,
        grid=(GRID,),
        in_specs=[_bspec((BN, 32)), _bspec((8, 32), True),
                  _bspec((1, 32), True), _bspec((1, 32), True),
                  _bspec((32, TF), True), _bspec((32, TF), True)],
        out_specs=[_bspec((BN, 32)), _bspec((BN, TF)), _bspec((BN, TF))],
        out_shape=[jax.ShapeDtypeStruct((N, 32), jnp.float32),
                   jax.ShapeDtypeStruct((N, TF), jnp.float32),
                   jax.ShapeDtypeStruct((N, TF), jnp.float32)],
    )(out_prev, acc, gamma, beta, wd, ws)


def _node_stage(h, A, S1, S2, MN, MX, degf, scal, wx, bd, bpost, wlin, blin):
    return pl.pallas_call(
        _k_node,
        grid=(GRID,),
        in_specs=[_bspec((BN, 32)), _bspec((BN, TF)), _bspec((BN, TF)),
                  _bspec((BN, TF)), _bspec((BN, TF)), _bspec((BN, TF)),
                  _bspec((BN, 1)), _bspec((1, 1), True),
                  _bspec((32, 32), True), _bspec((512, 96), True),
                  _bspec((1, 32), True), _bspec((32, 32), True),
                  _bspec((1, 32), True)],
        out_specs=[_bspec((BN, 32)), _bspec((8, 32), True)],
        out_shape=[jax.ShapeDtypeStruct((N, 32), jnp.float32),
                   jax.ShapeDtypeStruct((8, 32), jnp.float32)],
    )(h, A, S1, S2, MN, MX, degf, scal, wx, bd, bpost, wlin, blin)


def _final_stage(out3, acc, gamma, beta, batch2, w1, b1, w2, b2, w3, b3):
    res, _ = pl.pallas_call(
        _k_final,
        grid=(GRID,),
        in_specs=[_bspec((BN, 32)), _bspec((8, 32), True),
                  _bspec((1, 32), True), _bspec((1, 32), True),
                  _bspec((BN, 1)),
                  _bspec((32, 56), True), _bspec((1, 56), True),
                  _bspec((56, 32), True), _bspec((1, 32), True),
                  _bspec((32, 8), True), _bspec((1, 8), True)],
        out_specs=[_bspec((G, 8), True), _bspec((G, 32), True)],
        out_shape=[jax.ShapeDtypeStruct((G, 8), jnp.float32),
                   jax.ShapeDtypeStruct((G, 32), jnp.float32)],
    )(out3, acc, gamma, beta, batch2, w1, b1, w2, b2, w3, b3)
    return res


def _edge_stage(Bt, Ttab, src, dst, attr):
    w = Bt[src] + Ttab[attr]
    S1 = jax.ops.segment_sum(w, dst, num_segments=N)
    S2 = jax.ops.segment_sum(w * w, dst, num_segments=N)
    MN = jax.ops.segment_min(w, dst, num_segments=N)
    MX = jax.ops.segment_max(w, dst, num_segments=N)
    return S1, S2, jnp.nan_to_num(MN, posinf=0.0), jnp.nan_to_num(MX, neginf=0.0)


def kernel(params, x, edge_index, edge_attr, batch):
    w = _prep_weights(params)
    src = edge_index[0]
    dst = edge_index[1]
    deg = jax.ops.segment_sum(jnp.ones((E,), jnp.float32), dst, num_segments=N)
    degf = deg.reshape(N, 1)
    avg_log = jnp.mean(jnp.log(deg + 1.0))
    scal = avg_log.reshape(1, 1)

    x2 = x.reshape(N, 1).astype(jnp.int32)
    batch2 = batch.reshape(N, 1).astype(jnp.int32)

    h, A, Bt = _embed_proj(x2, w['emb'], w['Wd0'], w['Ws0'])
    out = None
    acc = None
    for l in range(L):
        if l > 0:
            h, A, Bt = _bn_proj(out, acc, w['gamma%d' % (l - 1)],
                                w['beta%d' % (l - 1)],
                                w['Wd%d' % l], w['Ws%d' % l])
        S1, S2, MN, MX = _edge_stage(Bt, w['Ttab%d' % l], src, dst, edge_attr)
        out, acc = _node_stage(h, A, S1, S2, MN, MX, degf, scal,
                               w['Wx%d' % l], w['BD%d' % l], w['bpost%d' % l],
                               w['Wlin%d' % l], w['blin%d' % l])
    res = _final_stage(out, acc, w['gamma3'], w['beta3'], batch2,
                       w['W1p'], w['b1p'], w['W2p'], w['b2p'],
                       w['W3p'], w['b3p'])
    return res[:, 0:1]
